# baseline (speedup 1.0000x reference)
import functools

import jax
import jax.numpy as jnp
from jax import lax
from jax.experimental import pallas as pl
from jax.experimental.pallas import tpu as pltpu
from jax.experimental.pallas import tpu_sc as plsc

N_NODES = 10000
D = 128
N_EDGES = 320000

NC = 2
NS = 16
NW = NC * NS
B = 128
NF = 2
NBLK = -(-N_EDGES // (NW * B * NF)) * NF
E_PAD = NW * NBLK * B
N_ACC = 10240
ROWS_PER_TILE_ACC = N_ACC // NS


def _sc_body(idx_hbm, source_hbm, zeros_hbm, partial_hbm,
             islot0, islot1, rows0, rows1, acc_sh,
             isem0, isem1, gsem0, gsem1):
    rows_bufs = (rows0, rows1)
    gsems = (gsem0, gsem1)
    islots = (islot0, islot1)
    isems = (isem0, isem1)
    c = lax.axis_index("c")
    s = lax.axis_index("s")
    wid = s * NC + c
    my_idx = idx_hbm.at[wid]

    pltpu.async_copy(my_idx.at[0], islots[0], isems[0])

    acc_base = s * ROWS_PER_TILE_ACC
    tile_rows = pl.ds(acc_base, ROWS_PER_TILE_ACC)
    pltpu.sync_copy(zeros_hbm.at[tile_rows], acc_sh.at[tile_rows])
    plsc.subcore_barrier()

    NG = NBLK // NF

    def wait_rows(sl, b):
        pltpu.make_async_copy(source_hbm.at[sl.at[b]], rows_bufs[b],
                              gsems[b]).wait()

    pltpu.make_async_copy(my_idx.at[0], islots[0], isems[0]).wait()
    pltpu.async_copy(my_idx.at[1], islots[1], isems[1])
    for b in range(NF):
        pltpu.async_copy(source_hbm.at[islots[0].at[b]], rows_bufs[b],
                         gsems[b])

    def body(gg, carry):
        for p in range(2):
            g = gg * 2 + p
            sl = islots[p]
            nsl = islots[(p + 1) % 2]
            nsem = isems[(p + 1) % 2]
            for b in range(NF):
                wait_rows(sl, b)
                pltpu.sync_copy(rows_bufs[b], acc_sh.at[sl.at[NF + b]],
                                add=True)
                if b == 0:
                    if p == 0:
                        pltpu.make_async_copy(my_idx.at[0], nsl,
                                              nsem).wait()
                    else:
                        @pl.when(g + 1 < NG)
                        def _():
                            pltpu.make_async_copy(my_idx.at[0], nsl,
                                                  nsem).wait()
                if p == 0:
                    pltpu.async_copy(source_hbm.at[nsl.at[b]],
                                     rows_bufs[b], gsems[b])
                else:
                    @pl.when(g + 1 < NG)
                    def _():
                        pltpu.async_copy(source_hbm.at[nsl.at[b]],
                                         rows_bufs[b], gsems[b])
            @pl.when(g + 2 < NG)
            def _():
                pltpu.async_copy(my_idx.at[g + 2], sl, isems[p])
        return carry

    lax.fori_loop(0, NG // 2, body, 0)
    plsc.subcore_barrier()

    pltpu.sync_copy(acc_sh.at[tile_rows], partial_hbm.at[c].at[tile_rows])


_sc_partial = functools.partial(
    pl.kernel,
    out_type=jax.ShapeDtypeStruct((NC, N_ACC, D), jnp.float32),
    mesh=plsc.VectorSubcoreMesh(core_axis_name="c", subcore_axis_name="s"),
    scratch_types=[
        pltpu.VMEM((2 * NF, B), jnp.int32),
        pltpu.VMEM((2 * NF, B), jnp.int32),
        pltpu.VMEM((B, D), jnp.float32),
        pltpu.VMEM((B, D), jnp.float32),
        pltpu.VMEM_SHARED((N_ACC, D), jnp.float32),
        pltpu.SemaphoreType.DMA,
        pltpu.SemaphoreType.DMA,
        pltpu.SemaphoreType.DMA,
        pltpu.SemaphoreType.DMA,
    ],
)(_sc_body)


def _combine_body(t_ref, p0_ref, p1_ref, o_ref):
    o_ref[...] = t_ref[...] + p0_ref[...] + p1_ref[...]


def _combine(target, p0, p1):
    blk = 1000
    grid = N_NODES // blk
    spec = pl.BlockSpec((blk, D), lambda i: (i, 0))
    return pl.pallas_call(
        _combine_body,
        grid=(grid,),
        in_specs=[spec, spec, spec],
        out_specs=spec,
        out_shape=jax.ShapeDtypeStruct((N_NODES, D), jnp.float32),
    )(target, p0, p1)


@jax.jit
def kernel(source, target, edge_index):
    src = edge_index[0].astype(jnp.int32)
    dst = edge_index[1].astype(jnp.int32)
    pad = E_PAD - N_EDGES
    pad_src = (jnp.arange(pad, dtype=jnp.int32) * 97) % N_NODES
    pad_dst = N_NODES + (jnp.arange(pad, dtype=jnp.int32) % (N_ACC - N_NODES))
    src_p = jnp.concatenate([src, pad_src]).reshape(NW, NBLK // NF, NF, B)
    dst_p = jnp.concatenate([dst, pad_dst]).reshape(NW, NBLK // NF, NF, B)
    idx_p = jnp.concatenate([src_p, dst_p], axis=2)
    zeros = jnp.zeros((N_ACC, D), jnp.float32)
    partial = _sc_partial(idx_p, source, zeros)
    return _combine(target, partial[0], partial[1])


# speedup vs baseline: 1.0277x; 1.0277x over previous
import functools

import jax
import jax.numpy as jnp
from jax import lax
from jax.experimental import pallas as pl
from jax.experimental.pallas import tpu as pltpu
from jax.experimental.pallas import tpu_sc as plsc

N_NODES = 10000
D = 128
N_EDGES = 320000

NC = 2
NS = 16
NW = NC * NS
B = 128
NF = 2
NBLK = -(-N_EDGES // (NW * B * NF)) * NF
E_PAD = NW * NBLK * B
N_ACC = 10240
ROWS_PER_TILE_ACC = N_ACC // NS


def _sc_body(idx_hbm, source_hbm, partial_hbm,
             islot0, islot1, rows0, rows1, zrow_v, acc_sh,
             isem0, isem1, gsem0, gsem1):
    rows_bufs = (rows0, rows1)
    gsems = (gsem0, gsem1)
    islots = (islot0, islot1)
    isems = (isem0, isem1)
    c = lax.axis_index("c")
    s = lax.axis_index("s")
    wid = s * NC + c
    my_idx = idx_hbm.at[wid]

    pltpu.async_copy(my_idx.at[0], islots[0], isems[0])

    zero = jnp.zeros((16,), jnp.float32)
    for i in range(16):
        for j in range(D // 16):
            zrow_v[i, pl.ds(j * 16, 16)] = zero

    acc_base = s * ROWS_PER_TILE_ACC

    def zbody(i, carry):
        pltpu.sync_copy(zrow_v, acc_sh.at[pl.ds(acc_base + i * 16, 16)])
        return carry

    lax.fori_loop(0, ROWS_PER_TILE_ACC // 16, zbody, 0)
    plsc.subcore_barrier()

    NG = NBLK // NF

    def wait_rows(sl, b):
        pltpu.make_async_copy(source_hbm.at[sl.at[b]], rows_bufs[b],
                              gsems[b]).wait()

    pltpu.make_async_copy(my_idx.at[0], islots[0], isems[0]).wait()
    pltpu.async_copy(my_idx.at[1], islots[1], isems[1])
    for b in range(NF):
        pltpu.async_copy(source_hbm.at[islots[0].at[b]], rows_bufs[b],
                         gsems[b])

    def body(gg, carry):
        for p in range(2):
            g = gg * 2 + p
            sl = islots[p]
            nsl = islots[(p + 1) % 2]
            nsem = isems[(p + 1) % 2]
            for b in range(NF):
                wait_rows(sl, b)
                pltpu.sync_copy(rows_bufs[b], acc_sh.at[sl.at[NF + b]],
                                add=True)
                if b == 0:
                    if p == 0:
                        pltpu.make_async_copy(my_idx.at[0], nsl,
                                              nsem).wait()
                    else:
                        @pl.when(g + 1 < NG)
                        def _():
                            pltpu.make_async_copy(my_idx.at[0], nsl,
                                                  nsem).wait()
                if p == 0:
                    pltpu.async_copy(source_hbm.at[nsl.at[b]],
                                     rows_bufs[b], gsems[b])
                else:
                    @pl.when(g + 1 < NG)
                    def _():
                        pltpu.async_copy(source_hbm.at[nsl.at[b]],
                                         rows_bufs[b], gsems[b])
            @pl.when(g + 2 < NG)
            def _():
                pltpu.async_copy(my_idx.at[g + 2], sl, isems[p])
        return carry

    lax.fori_loop(0, NG // 2, body, 0)
    plsc.subcore_barrier()

    pltpu.sync_copy(acc_sh.at[pl.ds(acc_base, ROWS_PER_TILE_ACC)],
                    partial_hbm.at[c].at[pl.ds(acc_base, ROWS_PER_TILE_ACC)])


_sc_partial = functools.partial(
    pl.kernel,
    out_type=jax.ShapeDtypeStruct((NC, N_ACC, D), jnp.float32),
    mesh=plsc.VectorSubcoreMesh(core_axis_name="c", subcore_axis_name="s"),
    scratch_types=[
        pltpu.VMEM((2 * NF, B), jnp.int32),
        pltpu.VMEM((2 * NF, B), jnp.int32),
        pltpu.VMEM((B, D), jnp.float32),
        pltpu.VMEM((B, D), jnp.float32),
        pltpu.VMEM((16, D), jnp.float32),
        pltpu.VMEM_SHARED((N_ACC, D), jnp.float32),
        pltpu.SemaphoreType.DMA,
        pltpu.SemaphoreType.DMA,
        pltpu.SemaphoreType.DMA,
        pltpu.SemaphoreType.DMA,
    ],
)(_sc_body)


def _combine_body(t_ref, p0_ref, p1_ref, o_ref):
    o_ref[...] = t_ref[...] + p0_ref[...] + p1_ref[...]


def _combine(target, p0, p1):
    blk = 1000
    grid = N_NODES // blk
    spec = pl.BlockSpec((blk, D), lambda i: (i, 0))
    return pl.pallas_call(
        _combine_body,
        grid=(grid,),
        in_specs=[spec, spec, spec],
        out_specs=spec,
        out_shape=jax.ShapeDtypeStruct((N_NODES, D), jnp.float32),
    )(target, p0, p1)


@jax.jit
def kernel(source, target, edge_index):
    src = edge_index[0].astype(jnp.int32)
    dst = edge_index[1].astype(jnp.int32)
    pad = E_PAD - N_EDGES
    pad_src = (jnp.arange(pad, dtype=jnp.int32) * 97) % N_NODES
    pad_dst = N_NODES + (jnp.arange(pad, dtype=jnp.int32) % (N_ACC - N_NODES))
    src_p = jnp.concatenate([src, pad_src]).reshape(NW, NBLK // NF, NF, B)
    dst_p = jnp.concatenate([dst, pad_dst]).reshape(NW, NBLK // NF, NF, B)
    idx_p = jnp.concatenate([src_p, dst_p], axis=2)
    partial = _sc_partial(idx_p, source)
    return _combine(target, partial[0], partial[1])
